# Initial kernel scaffold; baseline (speedup 1.0000x reference)
import jax
import jax.numpy as jnp
from jax.experimental import pallas as pl


def kernel(x, last_update, edge_index, t, msg, w_time, b_time, Wq, bq, Wk, bk, Wv, bv, We, Wskip, bskip):
    raise NotImplementedError("write your pallas kernel here")


# speedup vs baseline: 7.9585x; 7.9585x over previous
import functools

import jax
import jax.numpy as jnp
from jax import lax
from jax.experimental import pallas as pl
from jax.experimental.pallas import tpu as pltpu
from jax.experimental.pallas import tpu_sc as plsc

_N = 10000
_E = 320000
_D = 128
_H = 8
_C = 16
_TD = 32
_MSG = 16
_HC = _H * _C
_TW = 256
_NC = 2
_NS = 16
_NW = _NC * _NS
_EW = _E // _NW
_CH = 80
_NCH = _EW // _CH
_NB = 1000
_BE = 2000


def _dot(a, b):
    return lax.dot_general(a, b, (((1,), (0,)), ((), ())),
                           precision=lax.Precision.HIGHEST,
                           preferred_element_type=jnp.float32)


def _proj_body(x_ref, wqt, wkt, wvt, wst, bq, bk, bv, bs,
               q_ref, tsrc_ref, skip_ref):
    xb = x_ref[...]
    q_ref[...] = _dot(xb, wqt[...]) + bq[...]
    tsrc_ref[:, 0:128] = _dot(xb, wkt[...]) + bk[...]
    tsrc_ref[:, 128:256] = _dot(xb, wvt[...]) + bv[...]
    skip_ref[...] = _dot(xb, wst[...]) + bs[...]


def _stage1(x, wqt, wkt, wvt, wst, bq, bk, bv, bs):
    r = 2000
    w_spec = pl.BlockSpec((_D, _HC), lambda i: (0, 0))
    b_spec = pl.BlockSpec((1, _HC), lambda i: (0, 0))
    return pl.pallas_call(
        _proj_body,
        grid=(_N // r,),
        in_specs=[pl.BlockSpec((r, _D), lambda i: (i, 0)),
                  w_spec, w_spec, w_spec, w_spec,
                  b_spec, b_spec, b_spec, b_spec],
        out_specs=[pl.BlockSpec((r, _HC), lambda i: (i, 0)),
                   pl.BlockSpec((r, _TW), lambda i: (i, 0)),
                   pl.BlockSpec((r, _HC), lambda i: (i, 0))],
        out_shape=[jax.ShapeDtypeStruct((_N, _HC), jnp.float32),
                   jax.ShapeDtypeStruct((_N, _TW), jnp.float32),
                   jax.ShapeDtypeStruct((_N, _HC), jnp.float32)],
    )(x, wqt, wkt, wvt, wst, bq, bk, bv, bs)


def _gather_body(q_hbm, tsrc_hbm, lu_hbm, src_hbm, dst_hbm,
                 qd_hbm, gsrc_hbm, lus_hbm,
                 sidx, didx, qbuf, sbuf, lubuf, lu_v, sem1, sem2):
    c = lax.axis_index("c")
    s = lax.axis_index("s")
    base = (c * _NS + s) * _EW
    pltpu.sync_copy(lu_hbm, lu_v)

    def chunk(j, carry):
        off = base + j * _CH
        pltpu.sync_copy(src_hbm.at[pl.ds(off, _CH)], sidx)
        pltpu.sync_copy(dst_hbm.at[pl.ds(off, _CH)], didx)
        cp1 = pltpu.async_copy(tsrc_hbm.at[sidx], sbuf, sem1)
        cp2 = pltpu.async_copy(q_hbm.at[didx], qbuf, sem2)
        for i in range(_CH // 16):
            iv = sidx[pl.ds(i * 16, 16)]
            lubuf[pl.ds(i * 16, 16)] = plsc.load_gather(lu_v, [iv])
        cp1.wait()
        cp2.wait()
        pltpu.sync_copy(sbuf, gsrc_hbm.at[pl.ds(off, _CH)])
        pltpu.sync_copy(qbuf, qd_hbm.at[pl.ds(off, _CH)])
        pltpu.sync_copy(lubuf, lus_hbm.at[pl.ds(off, _CH)])
        return carry

    lax.fori_loop(0, _NCH, chunk, 0)


@functools.lru_cache(maxsize=1)
def _gather_kernel():
    return pl.kernel(
        _gather_body,
        out_type=[jax.ShapeDtypeStruct((_E, _HC), jnp.float32),
                  jax.ShapeDtypeStruct((_E, _TW), jnp.float32),
                  jax.ShapeDtypeStruct((_E,), jnp.float32)],
        mesh=plsc.VectorSubcoreMesh(core_axis_name="c", subcore_axis_name="s"),
        compiler_params=pltpu.CompilerParams(needs_layout_passes=False),
        scratch_types=[pltpu.VMEM((_CH,), jnp.int32),
                       pltpu.VMEM((_CH,), jnp.int32),
                       pltpu.VMEM((_CH, _HC), jnp.float32),
                       pltpu.VMEM((_CH, _TW), jnp.float32),
                       pltpu.VMEM((_CH,), jnp.float32),
                       pltpu.VMEM((_N,), jnp.float32),
                       pltpu.SemaphoreType.DMA,
                       pltpu.SemaphoreType.DMA])


def _edge_body(qd_ref, gsrc_ref, lus_ref, t_ref, msg_ref, wt_ref, bt_ref,
               wet_t, wet_m, outj_ref, ex16_ref):
    qd = qd_ref[...]
    ks = gsrc_ref[:, 0:128]
    vs = gsrc_ref[:, 128:256]
    rel = lus_ref[...] - t_ref[...]
    enc = jnp.cos(rel * wt_ref[...] + bt_ref[...])
    e = _dot(enc, wet_t[...]) + _dot(msg_ref[...], wet_m[...])
    sel = (lax.broadcasted_iota(jnp.int32, (128, 8), 0) // 16
           == lax.broadcasted_iota(jnp.int32, (128, 8), 1)).astype(jnp.float32)
    alpha = _dot(qd * (ks + e), sel) * 0.25
    ex = jnp.exp(alpha)
    sel_t = (lax.broadcasted_iota(jnp.int32, (8, 128), 0)
             == lax.broadcasted_iota(jnp.int32, (8, 128), 1) // 16).astype(jnp.float32)
    exb = _dot(ex, sel_t)
    outj_ref[...] = (vs + e) * exb
    pad = (lax.broadcasted_iota(jnp.int32, (8, 16), 0)
           == lax.broadcasted_iota(jnp.int32, (8, 16), 1)).astype(jnp.float32)
    ex16_ref[...] = _dot(ex, pad)


def _stage3(qd, gsrc, lus, t, msg, wt, bt, wet_t, wet_m):
    b = 2000
    return pl.pallas_call(
        _edge_body,
        grid=(_E // b,),
        in_specs=[pl.BlockSpec((b, _HC), lambda i: (i, 0)),
                  pl.BlockSpec((b, _TW), lambda i: (i, 0)),
                  pl.BlockSpec((b, 1), lambda i: (i, 0)),
                  pl.BlockSpec((b, 1), lambda i: (i, 0)),
                  pl.BlockSpec((b, _MSG), lambda i: (i, 0)),
                  pl.BlockSpec((1, _TD), lambda i: (0, 0)),
                  pl.BlockSpec((1, _TD), lambda i: (0, 0)),
                  pl.BlockSpec((_TD, _HC), lambda i: (0, 0)),
                  pl.BlockSpec((_MSG, _HC), lambda i: (0, 0))],
        out_specs=[pl.BlockSpec((b, _HC), lambda i: (i, 0)),
                   pl.BlockSpec((b, 16), lambda i: (i, 0))],
        out_shape=[jax.ShapeDtypeStruct((_E, _HC), jnp.float32),
                   jax.ShapeDtypeStruct((_E, 16), jnp.float32)],
    )(qd, gsrc, lus, t, msg, wt, bt, wet_t, wet_m)


def _segsum_body(dstr_ref, outj_ref, ex16_ref, acc_ref, den_ref):
    i = pl.program_id(1)

    @pl.when(i == 0)
    def _init():
        acc_ref[...] = jnp.zeros_like(acc_ref)
        den_ref[...] = jnp.zeros_like(den_ref)

    j = pl.program_id(0)
    rows = j * _NB + lax.broadcasted_iota(jnp.int32, (_NB, _BE), 0)
    mask = (rows == dstr_ref[0]).astype(jnp.bfloat16)
    oj = outj_ref[...].astype(jnp.bfloat16)
    exb = ex16_ref[...].astype(jnp.bfloat16)
    mm = lambda a, b: lax.dot_general(a, b, (((1,), (0,)), ((), ())),
                                      preferred_element_type=jnp.float32)
    acc_ref[...] += mm(mask, oj)
    den_ref[...] += mm(mask, exb)


def _stage4(dstr, outj, ex16):
    return pl.pallas_call(
        _segsum_body,
        grid=(_N // _NB, _E // _BE),
        in_specs=[pl.BlockSpec((1, 1, _BE), lambda j, i: (i, 0, 0)),
                  pl.BlockSpec((_BE, _HC), lambda j, i: (i, 0)),
                  pl.BlockSpec((_BE, 16), lambda j, i: (i, 0))],
        out_specs=[pl.BlockSpec((_NB, _HC), lambda j, i: (j, 0)),
                   pl.BlockSpec((_NB, 16), lambda j, i: (j, 0))],
        out_shape=[jax.ShapeDtypeStruct((_N, _HC), jnp.float32),
                   jax.ShapeDtypeStruct((_N, 16), jnp.float32)],
    )(dstr, outj, ex16)


def _final_body(acc_ref, den_ref, skip_ref, out_ref):
    acc = acc_ref[...]
    den = den_ref[...]
    bmat = (lax.broadcasted_iota(jnp.int32, (16, 128), 0)
            == lax.broadcasted_iota(jnp.int32, (16, 128), 1) // 16).astype(jnp.float32)
    denb = _dot(den, bmat)
    out_ref[...] = jnp.where(denb > 0, acc / denb, 0.0) + skip_ref[...]


def _stage5(acc, den, skip):
    r = 2000
    return pl.pallas_call(
        _final_body,
        grid=(_N // r,),
        in_specs=[pl.BlockSpec((r, _HC), lambda i: (i, 0)),
                  pl.BlockSpec((r, 16), lambda i: (i, 0)),
                  pl.BlockSpec((r, _HC), lambda i: (i, 0))],
        out_specs=pl.BlockSpec((r, _HC), lambda i: (i, 0)),
        out_shape=jax.ShapeDtypeStruct((_N, _HC), jnp.float32),
    )(acc, den, skip)


def kernel(x, last_update, edge_index, t, msg, w_time, b_time,
           Wq, bq, Wk, bk, Wv, bv, We, Wskip, bskip):
    src = edge_index[0]
    dst = edge_index[1]
    q, tsrc, skip = _stage1(x, Wq.T, Wk.T, Wv.T, Wskip.T,
                            bq[None, :], bk[None, :], bv[None, :], bskip[None, :])
    qd, gsrc, lus = _gather_kernel()(q, tsrc, last_update, src, dst)
    wet = We.T
    outj, ex16 = _stage3(qd, gsrc, lus[:, None], t[:, None], msg,
                         w_time[:, 0][None, :], b_time[None, :],
                         wet[:_TD], wet[_TD:])
    acc, den = _stage4(dst.reshape(_E // _BE, 1, _BE), outj, ex16)
    return _stage5(acc, den, skip)


# speedup vs baseline: 8.3740x; 1.0522x over previous
import functools

import jax
import jax.numpy as jnp
from jax import lax
from jax.experimental import pallas as pl
from jax.experimental.pallas import tpu as pltpu
from jax.experimental.pallas import tpu_sc as plsc

_N = 10000
_E = 320000
_D = 128
_H = 8
_C = 16
_TD = 32
_MSG = 16
_HC = _H * _C
_TW = 256
_NC = 2
_NS = 16
_NW = _NC * _NS
_EW = _E // _NW
_CH = 80
_NCH = _EW // _CH
_NB = 1000
_BE = 2000


def _dot(a, b):
    return lax.dot_general(a, b, (((1,), (0,)), ((), ())),
                           precision=lax.Precision.HIGHEST,
                           preferred_element_type=jnp.float32)


def _proj_body(x_ref, wqt, wkt, wvt, wst, bq, bk, bv, bs,
               q_ref, tsrc_ref, skip_ref):
    xb = x_ref[...]
    q_ref[...] = _dot(xb, wqt[...]) + bq[...]
    tsrc_ref[:, 0:128] = _dot(xb, wkt[...]) + bk[...]
    tsrc_ref[:, 128:256] = _dot(xb, wvt[...]) + bv[...]
    skip_ref[...] = _dot(xb, wst[...]) + bs[...]


def _stage1(x, wqt, wkt, wvt, wst, bq, bk, bv, bs):
    r = 2000
    w_spec = pl.BlockSpec((_D, _HC), lambda i: (0, 0))
    b_spec = pl.BlockSpec((1, _HC), lambda i: (0, 0))
    return pl.pallas_call(
        _proj_body,
        grid=(_N // r,),
        in_specs=[pl.BlockSpec((r, _D), lambda i: (i, 0)),
                  w_spec, w_spec, w_spec, w_spec,
                  b_spec, b_spec, b_spec, b_spec],
        out_specs=[pl.BlockSpec((r, _HC), lambda i: (i, 0)),
                   pl.BlockSpec((r, _TW), lambda i: (i, 0)),
                   pl.BlockSpec((r, _HC), lambda i: (i, 0))],
        out_shape=[jax.ShapeDtypeStruct((_N, _HC), jnp.float32),
                   jax.ShapeDtypeStruct((_N, _TW), jnp.float32),
                   jax.ShapeDtypeStruct((_N, _HC), jnp.float32)],
    )(x, wqt, wkt, wvt, wst, bq, bk, bv, bs)


def _gather_body(q_hbm, tsrc_hbm, lu_hbm, src_hbm, dst_hbm,
                 qd_hbm, gsrc_hbm, lus_hbm,
                 sidx, didx, qbuf, sbuf, lubuf, lu_v, sem1, sem2):
    c = lax.axis_index("c")
    s = lax.axis_index("s")
    base = (c * _NS + s) * _EW
    pltpu.sync_copy(lu_hbm, lu_v)

    def chunk(j, carry):
        off = base + j * _CH
        pltpu.sync_copy(src_hbm.at[pl.ds(off, _CH)], sidx)
        pltpu.sync_copy(dst_hbm.at[pl.ds(off, _CH)], didx)
        cp1 = pltpu.async_copy(tsrc_hbm.at[sidx], sbuf, sem1)
        cp2 = pltpu.async_copy(q_hbm.at[didx], qbuf, sem2)
        for i in range(_CH // 16):
            iv = sidx[pl.ds(i * 16, 16)]
            lubuf[pl.ds(i * 16, 16)] = plsc.load_gather(lu_v, [iv])
        cp1.wait()
        cp2.wait()
        pltpu.sync_copy(sbuf, gsrc_hbm.at[pl.ds(off, _CH)])
        pltpu.sync_copy(qbuf, qd_hbm.at[pl.ds(off, _CH)])
        pltpu.sync_copy(lubuf, lus_hbm.at[pl.ds(off, _CH)])
        return carry

    lax.fori_loop(0, _NCH, chunk, 0)


@functools.lru_cache(maxsize=1)
def _gather_kernel():
    return pl.kernel(
        _gather_body,
        out_type=[jax.ShapeDtypeStruct((_E, _HC), jnp.float32),
                  jax.ShapeDtypeStruct((_E, _TW), jnp.float32),
                  jax.ShapeDtypeStruct((_E,), jnp.float32)],
        mesh=plsc.VectorSubcoreMesh(core_axis_name="c", subcore_axis_name="s"),
        compiler_params=pltpu.CompilerParams(needs_layout_passes=False),
        scratch_types=[pltpu.VMEM((_CH,), jnp.int32),
                       pltpu.VMEM((_CH,), jnp.int32),
                       pltpu.VMEM((_CH, _HC), jnp.float32),
                       pltpu.VMEM((_CH, _TW), jnp.float32),
                       pltpu.VMEM((_CH,), jnp.float32),
                       pltpu.VMEM((_N,), jnp.float32),
                       pltpu.SemaphoreType.DMA,
                       pltpu.SemaphoreType.DMA])


def _edge_body(qd_ref, gsrc_ref, lus_ref, t_ref, msg_ref, wt_ref, bt_ref,
               wet_t, wet_m, outj_ref, ex16_ref):
    qd = qd_ref[...]
    ks = gsrc_ref[:, 0:128]
    vs = gsrc_ref[:, 128:256]
    rel = lus_ref[...] - t_ref[...]
    enc = jnp.cos(rel * wt_ref[...] + bt_ref[...])
    e = _dot(enc, wet_t[...]) + _dot(msg_ref[...], wet_m[...])
    sel = (lax.broadcasted_iota(jnp.int32, (128, 8), 0) // 16
           == lax.broadcasted_iota(jnp.int32, (128, 8), 1)).astype(jnp.float32)
    alpha = _dot(qd * (ks + e), sel) * 0.25
    ex = jnp.exp(alpha)
    sel_t = (lax.broadcasted_iota(jnp.int32, (8, 128), 0)
             == lax.broadcasted_iota(jnp.int32, (8, 128), 1) // 16).astype(jnp.float32)
    exb = _dot(ex, sel_t)
    outj_ref[...] = (vs + e) * exb
    pad = (lax.broadcasted_iota(jnp.int32, (8, 16), 0)
           == lax.broadcasted_iota(jnp.int32, (8, 16), 1)).astype(jnp.float32)
    ex16_ref[...] = _dot(ex, pad)


def _stage3(qd, gsrc, lus, t, msg, wt, bt, wet_t, wet_m):
    b = 2000
    return pl.pallas_call(
        _edge_body,
        grid=(_E // b,),
        in_specs=[pl.BlockSpec((b, _HC), lambda i: (i, 0)),
                  pl.BlockSpec((b, _TW), lambda i: (i, 0)),
                  pl.BlockSpec((b, 1), lambda i: (i, 0)),
                  pl.BlockSpec((b, 1), lambda i: (i, 0)),
                  pl.BlockSpec((b, _MSG), lambda i: (i, 0)),
                  pl.BlockSpec((1, _TD), lambda i: (0, 0)),
                  pl.BlockSpec((1, _TD), lambda i: (0, 0)),
                  pl.BlockSpec((_TD, _HC), lambda i: (0, 0)),
                  pl.BlockSpec((_MSG, _HC), lambda i: (0, 0))],
        out_specs=[pl.BlockSpec((b, _HC), lambda i: (i, 0)),
                   pl.BlockSpec((b, 16), lambda i: (i, 0))],
        out_shape=[jax.ShapeDtypeStruct((_E, _HC), jnp.float32),
                   jax.ShapeDtypeStruct((_E, 16), jnp.float32)],
    )(qd, gsrc, lus, t, msg, wt, bt, wet_t, wet_m)


def _segsum_body(dstr_ref, outj_ref, ex16_ref, acc_ref, den_ref):
    i = pl.program_id(0)

    @pl.when(i == 0)
    def _init():
        acc_ref[...] = jnp.zeros_like(acc_ref)
        den_ref[...] = jnp.zeros_like(den_ref)

    oj = outj_ref[...].astype(jnp.bfloat16)
    exb = ex16_ref[...].astype(jnp.bfloat16)
    dstb = dstr_ref[0]
    mm = lambda a, b: lax.dot_general(a, b, (((1,), (0,)), ((), ())),
                                      preferred_element_type=jnp.float32)
    for j in range(_N // _NB):
        rows = j * _NB + lax.broadcasted_iota(jnp.int32, (_NB, _BE), 0)
        mask = (rows == dstb).astype(jnp.bfloat16)
        acc_ref[pl.ds(j * _NB, _NB), :] += mm(mask, oj)
        den_ref[pl.ds(j * _NB, _NB), :] += mm(mask, exb)


def _stage4(dstr, outj, ex16):
    return pl.pallas_call(
        _segsum_body,
        grid=(_E // _BE,),
        in_specs=[pl.BlockSpec((1, 1, _BE), lambda i: (i, 0, 0)),
                  pl.BlockSpec((_BE, _HC), lambda i: (i, 0)),
                  pl.BlockSpec((_BE, 16), lambda i: (i, 0))],
        out_specs=[pl.BlockSpec((_N, _HC), lambda i: (0, 0)),
                   pl.BlockSpec((_N, 16), lambda i: (0, 0))],
        out_shape=[jax.ShapeDtypeStruct((_N, _HC), jnp.float32),
                   jax.ShapeDtypeStruct((_N, 16), jnp.float32)],
    )(dstr, outj, ex16)


def _final_body(acc_ref, den_ref, skip_ref, out_ref):
    acc = acc_ref[...]
    den = den_ref[...]
    bmat = (lax.broadcasted_iota(jnp.int32, (16, 128), 0)
            == lax.broadcasted_iota(jnp.int32, (16, 128), 1) // 16).astype(jnp.float32)
    denb = _dot(den, bmat)
    out_ref[...] = jnp.where(denb > 0, acc / denb, 0.0) + skip_ref[...]


def _stage5(acc, den, skip):
    r = 2000
    return pl.pallas_call(
        _final_body,
        grid=(_N // r,),
        in_specs=[pl.BlockSpec((r, _HC), lambda i: (i, 0)),
                  pl.BlockSpec((r, 16), lambda i: (i, 0)),
                  pl.BlockSpec((r, _HC), lambda i: (i, 0))],
        out_specs=pl.BlockSpec((r, _HC), lambda i: (i, 0)),
        out_shape=jax.ShapeDtypeStruct((_N, _HC), jnp.float32),
    )(acc, den, skip)


def kernel(x, last_update, edge_index, t, msg, w_time, b_time,
           Wq, bq, Wk, bk, Wv, bv, We, Wskip, bskip):
    src = edge_index[0]
    dst = edge_index[1]
    q, tsrc, skip = _stage1(x, Wq.T, Wk.T, Wv.T, Wskip.T,
                            bq[None, :], bk[None, :], bv[None, :], bskip[None, :])
    qd, gsrc, lus = _gather_kernel()(q, tsrc, last_update, src, dst)
    wet = We.T
    outj, ex16 = _stage3(qd, gsrc, lus[:, None], t[:, None], msg,
                         w_time[:, 0][None, :], b_time[None, :],
                         wet[:_TD], wet[_TD:])
    acc, den = _stage4(dst.reshape(_E // _BE, 1, _BE), outj, ex16)
    return _stage5(acc, den, skip)
